# Initial kernel scaffold; baseline (speedup 1.0000x reference)
import jax
import jax.numpy as jnp
from jax.experimental import pallas as pl


def kernel(img_nchw, meta, w1, b1, w2, b2, w3, b3, w_img_t, w_meta, b_meta, w_meta_out, b_out):
    raise NotImplementedError("write your pallas kernel here")


# speedup vs baseline: 6.2140x; 6.2140x over previous
import jax
import jax.numpy as jnp
from jax.experimental import pallas as pl
from jax.experimental.pallas import tpu as pltpu


def _pool2x2(act_ref, h, w, c):
    m = (h * w) // 2
    pw = jnp.maximum(act_ref[pl.ds(0, m, 2), :], act_ref[pl.ds(1, m, 2), :])
    return jnp.max(pw.reshape(h // 2, 2, w // 2, c), axis=1).reshape(
        (h // 2) * (w // 2), c)


def _row_patches(x, hw, w, c):
    col = jax.lax.broadcasted_iota(jnp.int32, (hw, c), 0) % w
    zeros_row = jnp.zeros((1, c), x.dtype)
    left = jnp.concatenate([zeros_row, x[: hw - 1, :]], axis=0)
    left = jnp.where(col == 0, jnp.bfloat16(0), left)
    right = jnp.concatenate([x[1:, :], zeros_row], axis=0)
    right = jnp.where(col == w - 1, jnp.bfloat16(0), right)
    return jnp.concatenate([left, x, right], axis=1)


def kernel(img_nchw, meta, w1, b1, w2, b2, w3, b3, w_img_t,
           w_meta, b_meta, w_meta_out, b_out):
    B, Cin, H, W = img_nchw.shape
    C1 = w1.shape[1]
    C2 = w2.shape[1]
    C3 = w3.shape[1]
    H2, W2 = H // 2, W // 2
    H3, W3 = H // 4, W // 4
    HW1, HW2, HW3 = H * W, H2 * W2, H3 * W3
    R = (H // 8) * (W // 8)
    NC = w_img_t.shape[0]
    KC1 = w1.shape[0]

    x = jnp.transpose(img_nchw, (0, 2, 3, 1)).astype(jnp.bfloat16)
    xp = jnp.pad(x, ((0, 0), (1, 1), (1, 1), (0, 0)))
    taps = [xp[:, dh:dh + H, dw:dw + W, :]
            for dh in range(3) for dw in range(3)]
    p1 = jnp.concatenate(taps, axis=-1).reshape(B, HW1, KC1)

    w1b = w1.astype(jnp.bfloat16)
    w2r = w2.reshape(3, 3 * C1, C2).astype(jnp.bfloat16)
    w3r = w3.reshape(3, 3 * C2, C3).astype(jnp.bfloat16)
    meta3 = meta.reshape(B, 1, meta.shape[1])

    def body(p1_ref, meta_ref, w1_ref, b1_ref, w2_ref, b2_ref, w3_ref, b3_ref,
             wi_ref, wm_ref, bm_ref, wmo_ref, bo_ref, o_ref,
             act1_s, p2_s, act2_s, p3_s, act3_s):
        a1 = jnp.dot(p1_ref[0], w1_ref[...],
                     preferred_element_type=jnp.float32)
        act1_s[...] = jnp.maximum(a1 + b1_ref[...], 0.0)
        x2 = _pool2x2(act1_s, H, W, C1).astype(jnp.bfloat16)

        p2_s[pl.ds(0, W2), :] = jnp.zeros((W2, 3 * C1), jnp.bfloat16)
        p2_s[pl.ds(W2 + HW2, W2), :] = jnp.zeros((W2, 3 * C1), jnp.bfloat16)
        p2_s[pl.ds(W2, HW2), :] = _row_patches(x2, HW2, W2, C1)
        a2 = (jnp.dot(p2_s[pl.ds(0, HW2), :], w2_ref[0],
                      preferred_element_type=jnp.float32)
              + jnp.dot(p2_s[pl.ds(W2, HW2), :], w2_ref[1],
                        preferred_element_type=jnp.float32)
              + jnp.dot(p2_s[pl.ds(2 * W2, HW2), :], w2_ref[2],
                        preferred_element_type=jnp.float32))
        act2_s[...] = jnp.maximum(a2 + b2_ref[...], 0.0)
        x3 = _pool2x2(act2_s, H2, W2, C2).astype(jnp.bfloat16)

        p3_s[pl.ds(0, W3), :] = jnp.zeros((W3, 3 * C2), jnp.bfloat16)
        p3_s[pl.ds(W3 + HW3, W3), :] = jnp.zeros((W3, 3 * C2), jnp.bfloat16)
        p3_s[pl.ds(W3, HW3), :] = _row_patches(x3, HW3, W3, C2)
        a3 = (jnp.dot(p3_s[pl.ds(0, HW3), :], w3_ref[0],
                      preferred_element_type=jnp.float32)
              + jnp.dot(p3_s[pl.ds(W3, HW3), :], w3_ref[1],
                        preferred_element_type=jnp.float32)
              + jnp.dot(p3_s[pl.ds(2 * W3, HW3), :], w3_ref[2],
                        preferred_element_type=jnp.float32))
        act3_s[...] = jnp.maximum(a3 + b3_ref[...], 0.0)
        xf = _pool2x2(act3_s, H3, W3, C3)

        l0 = jnp.sum(wi_ref[0] * xf)
        l1 = jnp.sum(wi_ref[1] * xf)
        mo = jnp.maximum(
            jnp.dot(meta_ref[0], wm_ref[...],
                    preferred_element_type=jnp.float32) + bm_ref[...], 0.0)
        ml = jnp.dot(mo, wmo_ref[...], preferred_element_type=jnp.float32)
        il = jnp.concatenate([jnp.full((1, 1), l0, jnp.float32),
                              jnp.full((1, 1), l1, jnp.float32)], axis=1)
        o_ref[0] = ml + bo_ref[...] + il

    const2 = lambda b: (0, 0)
    const3 = lambda b: (0, 0, 0)
    out = pl.pallas_call(
        body,
        out_shape=jax.ShapeDtypeStruct((B, 1, NC), jnp.float32),
        grid=(B,),
        in_specs=[
            pl.BlockSpec((1, HW1, KC1), lambda b: (b, 0, 0)),
            pl.BlockSpec((1, 1, meta.shape[1]), lambda b: (b, 0, 0)),
            pl.BlockSpec(w1b.shape, const2),
            pl.BlockSpec(b1.shape, const2),
            pl.BlockSpec(w2r.shape, const3),
            pl.BlockSpec(b2.shape, const2),
            pl.BlockSpec(w3r.shape, const3),
            pl.BlockSpec(b3.shape, const2),
            pl.BlockSpec(w_img_t.shape, const3),
            pl.BlockSpec(w_meta.shape, const2),
            pl.BlockSpec(b_meta.shape, const2),
            pl.BlockSpec(w_meta_out.shape, const2),
            pl.BlockSpec(b_out.shape, const2),
        ],
        out_specs=pl.BlockSpec((1, 1, NC), lambda b: (b, 0, 0)),
        scratch_shapes=[
            pltpu.VMEM((HW1, C1), jnp.float32),
            pltpu.VMEM((HW2 + 2 * W2, 3 * C1), jnp.bfloat16),
            pltpu.VMEM((HW2, C2), jnp.float32),
            pltpu.VMEM((HW3 + 2 * W3, 3 * C2), jnp.bfloat16),
            pltpu.VMEM((HW3, C3), jnp.float32),
        ],
        compiler_params=pltpu.CompilerParams(
            dimension_semantics=("parallel",),
            vmem_limit_bytes=48 * 1024 * 1024),
    )(p1, meta3, w1b, b1, w2r, b2, w3r, b3, w_img_t,
      w_meta, b_meta, w_meta_out, b_out)
    return out.reshape(B, NC)
